# baseline (speedup 1.0000x reference)
import functools

import jax
import jax.numpy as jnp
from jax import lax
from jax.experimental import pallas as pl
from jax.experimental.pallas import tpu as pltpu
from jax.experimental.pallas import tpu_sc as plsc

N = 10000
E = 320000
F_IN = 32
H = 128
L = 6
A = 6158
P = 4

NC = 2
NS = 16
NW = NC * NS

K = 128
NBUF = 2
NCHUNK = 80
EPW = NCHUNK * K
EPAD = EPW * NW
NT = NCHUNK // NBUF
NPAD = 10240
RPT = NPAD // NS
CW = 128

def _sc_agg(h_hbm, src_hbm, dst_hbm, zeros_hbm, out_hbm,
            src_v, dst_v, rows_v, acc_sh, gsem):
    c = lax.axis_index("c")
    s = lax.axis_index("s")
    wid = s * NC + c

    @pl.when(s == 0)
    def _():
        pltpu.sync_copy(zeros_hbm, acc_sh)

    plsc.subcore_barrier()

    base = wid * EPW

    def body(g, carry):
        off = base + g * K
        pltpu.sync_copy(src_hbm.at[pl.ds(off, K)], src_v)
        pltpu.sync_copy(dst_hbm.at[pl.ds(off, K)], dst_v)
        pltpu.async_copy(h_hbm.at[src_v], rows_v, gsem).wait()
        pltpu.sync_copy(rows_v, acc_sh.at[dst_v], add=True)
        return carry

    lax.fori_loop(0, NCHUNK, body, 0)

    plsc.subcore_barrier()
    pltpu.sync_copy(acc_sh.at[pl.ds(s * RPT, RPT)],
                    out_hbm.at[c].at[pl.ds(s * RPT, RPT)])


def _sc_counts(dst_hbm, ones_hbm, zeros_hbm, out_hbm,
               dst_v, ones_v, acc_sh):
    c = lax.axis_index("c")
    s = lax.axis_index("s")
    wid = s * NC + c

    @pl.when(s == 0)
    def _():
        pltpu.sync_copy(zeros_hbm, acc_sh)

    pltpu.sync_copy(ones_hbm, ones_v)
    pltpu.sync_copy(dst_hbm.at[wid], dst_v)
    plsc.subcore_barrier()

    def body(g, carry):
        pltpu.sync_copy(ones_v, acc_sh.at[dst_v.at[g]], add=True)
        return carry

    lax.fori_loop(0, NCHUNK, body, 0)

    plsc.subcore_barrier()
    pltpu.sync_copy(acc_sh.at[pl.ds(s * RPT, RPT)],
                    out_hbm.at[c].at[pl.ds(s * RPT, RPT)])


@functools.lru_cache(maxsize=None)
def _sc_calls():
    mesh = plsc.VectorSubcoreMesh(core_axis_name="c", subcore_axis_name="s",
                                  num_cores=NC, num_subcores=NS)
    agg = pl.kernel(
        _sc_agg,
        mesh=mesh,
        out_type=jax.ShapeDtypeStruct((NC, NPAD, H), jnp.float32),
        scratch_types=[
            pltpu.VMEM((K,), jnp.int32),
            pltpu.VMEM((K,), jnp.int32),
            pltpu.VMEM((K, H), jnp.float32),
            pltpu.VMEM_SHARED((NPAD, H), jnp.float32),
            pltpu.SemaphoreType.DMA,
        ],
    )
    counts = pl.kernel(
        _sc_counts,
        mesh=mesh,
        out_type=jax.ShapeDtypeStruct((NC, NPAD, CW), jnp.float32),
        scratch_types=[
            pltpu.VMEM((NCHUNK, K), jnp.int32),
            pltpu.VMEM((K, CW), jnp.float32),
            pltpu.VMEM_SHARED((NPAD, CW), jnp.float32),
        ],
    )
    return agg, counts


def _dotT(a, b):
    return lax.dot_general(a, b, (((1,), (1,)), ((), ())),
                           preferred_element_type=jnp.float32)


def _ln(t, g, b):
    m = jnp.mean(t, axis=-1, keepdims=True)
    v = jnp.mean((t - m) ** 2, axis=-1, keepdims=True)
    return (t - m) * lax.rsqrt(v + 1e-5) * g + b


def _enc_body(x_ref, w_ref, b_ref, g_ref, be_ref, out_ref):
    t = _dotT(x_ref[...], w_ref[...]) + b_ref[...]
    out_ref[...] = jnp.maximum(_ln(t, g_ref[...], be_ref[...]), 0.0)


_enc_call = pl.pallas_call(
    _enc_body, out_shape=jax.ShapeDtypeStruct((N, H), jnp.float32))


def _layer_body(aggp_ref, cnt_ref, h_ref, wl_ref, bl_ref, wr_ref,
                g_ref, b_ref, out_ref):
    cnt = cnt_ref[0, 0:N, 0:1] + cnt_ref[1, 0:N, 0:1]
    inv = 1.0 / jnp.maximum(cnt, 1.0)
    agg = (aggp_ref[0, 0:N, :] + aggp_ref[1, 0:N, :]) * inv
    hx = h_ref[...]
    t = _dotT(agg, wl_ref[...]) + bl_ref[...] + _dotT(hx, wr_ref[...])
    out_ref[...] = hx + jnp.maximum(_ln(t, g_ref[...], b_ref[...]), 0.0)


_layer_call = pl.pallas_call(
    _layer_body, out_shape=jax.ShapeDtypeStruct((N, H), jnp.float32))


def _pool_body(h_ref, a1_ref, a1b_ref, a2_ref,
               pgp_ref, pgpb_ref, ph1_ref, ph1b_ref, ph2_ref, ph2b_ref,
               vh1_ref, vh1b_ref, vh2_ref, vh2b_ref, vh3_ref, vh3b_ref,
               pol_ref, val_ref):
    hx = h_ref[...]
    t = jnp.tanh(_dotT(hx, a1_ref[...]) + a1b_ref[...])
    scores = _dotT(t, a2_ref[...])
    m = jnp.max(scores)
    e = jnp.exp(scores - m)
    w = e * (1.0 / jnp.sum(e))
    gr = jnp.sum(hx * w, axis=0, keepdims=True)
    z = _dotT(gr, pgp_ref[...]) + pgpb_ref[...]
    p1 = jnp.maximum(_dotT(z, ph1_ref[...]) + ph1b_ref[...], 0.0)
    pol_ref[...] = _dotT(p1, ph2_ref[...]) + ph2b_ref[...]
    v1 = jnp.maximum(_dotT(gr, vh1_ref[...]) + vh1b_ref[...], 0.0)
    v2 = jnp.maximum(_dotT(v1, vh2_ref[...]) + vh2b_ref[...], 0.0)
    val_ref[...] = jnp.tanh(_dotT(v2, vh3_ref[...]) + vh3b_ref[...])


_APAD = 6272
_PPAD = 8

_pool_call = pl.pallas_call(
    _pool_body,
    out_shape=(jax.ShapeDtypeStruct((1, _APAD), jnp.float32),
               jax.ShapeDtypeStruct((1, _PPAD), jnp.float32)))


def kernel(x, edge_index, enc_W, enc_b, ln0_g, ln0_b, Wl, bl, Wr, lng, lnb,
           A1, a1b, A2, a2b, pgp_W, pgp_b, ph1_W, ph1_b, ph2_W, ph2_b,
           vh1_W, vh1_b, vh2_W, vh2_b, vh3_W, vh3_b):
    src = edge_index[0].astype(jnp.int32)
    dst = edge_index[1].astype(jnp.int32)
    pad = EPAD - E
    src_p = jnp.concatenate([src, jnp.zeros((pad,), jnp.int32)])
    dst_p = jnp.concatenate([dst, jnp.full((pad,), N, jnp.int32)])
    dst_p3 = dst_p.reshape(NW, NCHUNK, K)

    zeros_h = jnp.zeros((NPAD, H), jnp.float32)
    zeros_c = jnp.zeros((NPAD, CW), jnp.float32)
    ones_c = jnp.ones((K, CW), jnp.float32)

    b2 = lambda v: v.reshape(1, -1)

    sc_agg_call, sc_counts_call = _sc_calls()
    h = _enc_call(x, enc_W, b2(enc_b), b2(ln0_g), b2(ln0_b))
    cnt = sc_counts_call(dst_p3, ones_c, zeros_c)
    h, cnt = lax.optimization_barrier((h, cnt))

    for i in range(L):
        aggp = sc_agg_call(h, src_p, dst_p, zeros_h)
        h = _layer_call(aggp, cnt, h, Wl[i], b2(bl[i]), Wr[i],
                        b2(lng[i]), b2(lnb[i]))

    ph2_Wp = jnp.zeros((_APAD, H), jnp.float32).at[:A].set(ph2_W)
    ph2_bp = jnp.zeros((1, _APAD), jnp.float32).at[:, :A].set(b2(ph2_b))
    vh3_Wp = jnp.zeros((_PPAD, H // 2), jnp.float32).at[:P].set(vh3_W)
    vh3_bp = jnp.zeros((1, _PPAD), jnp.float32).at[:, :P].set(b2(vh3_b))

    pol_p, val_p = _pool_call(
        h, A1, b2(a1b), A2,
        pgp_W[:, :H], b2(pgp_b), ph1_W, b2(ph1_b), ph2_Wp, ph2_bp,
        vh1_W[:, :H], b2(vh1_b), vh2_W, b2(vh2_b), vh3_Wp, vh3_bp)

    return pol_p[:, :A], val_p[:, :P]


# speedup vs baseline: 1.4219x; 1.4219x over previous
import functools

import jax
import jax.numpy as jnp
from jax import lax
from jax.experimental import pallas as pl
from jax.experimental.pallas import tpu as pltpu
from jax.experimental.pallas import tpu_sc as plsc

N = 10000
E = 320000
F_IN = 32
H = 128
L = 6
A = 6158
P = 4

NC = 2
NS = 16
NW = NC * NS

K = 128
EPW = ((E + NW * K - 1) // (NW * K)) * K
EPAD = EPW * NW
NCHUNK = EPW // K
NPAD = 10240
RPT = NPAD // NS
CW = 128

def _sc_agg(h_hbm, src_hbm, dst_hbm, zeros_hbm, out_hbm,
            src_v, dst_v, rows_v, acc_sh, sem):
    c = lax.axis_index("c")
    s = lax.axis_index("s")
    wid = s * NC + c

    @pl.when(s == 0)
    def _():
        pltpu.sync_copy(zeros_hbm, acc_sh)

    plsc.subcore_barrier()

    base = wid * EPW

    def body(g, carry):
        off = base + g * K
        pltpu.sync_copy(src_hbm.at[pl.ds(off, K)], src_v)
        pltpu.sync_copy(dst_hbm.at[pl.ds(off, K)], dst_v)
        pltpu.async_copy(h_hbm.at[src_v], rows_v, sem).wait()
        pltpu.sync_copy(rows_v, acc_sh.at[dst_v], add=True)
        return carry

    lax.fori_loop(0, NCHUNK, body, 0)

    plsc.subcore_barrier()
    pltpu.sync_copy(acc_sh.at[pl.ds(s * RPT, RPT)],
                    out_hbm.at[c].at[pl.ds(s * RPT, RPT)])


def _sc_counts(dst_hbm, ones_hbm, zeros_hbm, out_hbm,
               dst_v, ones_v, acc_sh, sem):
    c = lax.axis_index("c")
    s = lax.axis_index("s")
    wid = s * NC + c

    @pl.when(s == 0)
    def _():
        pltpu.sync_copy(zeros_hbm, acc_sh)

    pltpu.async_copy(ones_hbm, ones_v, sem).wait()
    plsc.subcore_barrier()

    base = wid * EPW

    def body(g, carry):
        off = base + g * K
        pltpu.sync_copy(dst_hbm.at[pl.ds(off, K)], dst_v)
        pltpu.sync_copy(ones_v, acc_sh.at[dst_v], add=True)
        return carry

    lax.fori_loop(0, NCHUNK, body, 0)

    plsc.subcore_barrier()
    pltpu.sync_copy(acc_sh.at[pl.ds(s * RPT, RPT)],
                    out_hbm.at[c].at[pl.ds(s * RPT, RPT)])


@functools.lru_cache(maxsize=None)
def _sc_calls():
    mesh = plsc.VectorSubcoreMesh(core_axis_name="c", subcore_axis_name="s",
                                  num_cores=NC, num_subcores=NS)
    agg = pl.kernel(
        _sc_agg,
        mesh=mesh,
        out_type=jax.ShapeDtypeStruct((NC, NPAD, H), jnp.float32),
        scratch_types=[
            pltpu.VMEM((K,), jnp.int32),
            pltpu.VMEM((K,), jnp.int32),
            pltpu.VMEM((K, H), jnp.float32),
            pltpu.VMEM_SHARED((NPAD, H), jnp.float32),
            pltpu.SemaphoreType.DMA,
        ],
    )
    counts = pl.kernel(
        _sc_counts,
        mesh=mesh,
        out_type=jax.ShapeDtypeStruct((NC, NPAD, CW), jnp.float32),
        scratch_types=[
            pltpu.VMEM((K,), jnp.int32),
            pltpu.VMEM((K, CW), jnp.float32),
            pltpu.VMEM_SHARED((NPAD, CW), jnp.float32),
            pltpu.SemaphoreType.DMA,
        ],
    )
    return agg, counts


def _dotT(a, b):
    return lax.dot_general(a, b, (((1,), (1,)), ((), ())),
                           preferred_element_type=jnp.float32)


def _ln(t, g, b):
    m = jnp.mean(t, axis=-1, keepdims=True)
    v = jnp.mean((t - m) ** 2, axis=-1, keepdims=True)
    return (t - m) * lax.rsqrt(v + 1e-5) * g + b


def _enc_body(x_ref, w_ref, b_ref, g_ref, be_ref, out_ref):
    t = _dotT(x_ref[...], w_ref[...]) + b_ref[...]
    out_ref[...] = jnp.maximum(_ln(t, g_ref[...], be_ref[...]), 0.0)


_enc_call = pl.pallas_call(
    _enc_body, out_shape=jax.ShapeDtypeStruct((N, H), jnp.float32))


def _layer_body(aggp_ref, cnt_ref, h_ref, wl_ref, bl_ref, wr_ref,
                g_ref, b_ref, out_ref):
    cnt = cnt_ref[0, 0:N, 0:1] + cnt_ref[1, 0:N, 0:1]
    inv = 1.0 / jnp.maximum(cnt, 1.0)
    agg = (aggp_ref[0, 0:N, :] + aggp_ref[1, 0:N, :]) * inv
    hx = h_ref[...]
    t = _dotT(agg, wl_ref[...]) + bl_ref[...] + _dotT(hx, wr_ref[...])
    out_ref[...] = hx + jnp.maximum(_ln(t, g_ref[...], b_ref[...]), 0.0)


_layer_call = pl.pallas_call(
    _layer_body, out_shape=jax.ShapeDtypeStruct((N, H), jnp.float32))


def _pool_body(h_ref, a1_ref, a1b_ref, a2_ref,
               pgp_ref, pgpb_ref, ph1_ref, ph1b_ref, ph2_ref, ph2b_ref,
               vh1_ref, vh1b_ref, vh2_ref, vh2b_ref, vh3_ref, vh3b_ref,
               pol_ref, val_ref):
    hx = h_ref[...]
    t = jnp.tanh(_dotT(hx, a1_ref[...]) + a1b_ref[...])
    scores = _dotT(t, a2_ref[...])
    m = jnp.max(scores)
    e = jnp.exp(scores - m)
    w = e * (1.0 / jnp.sum(e))
    gr = jnp.sum(hx * w, axis=0, keepdims=True)
    z = _dotT(gr, pgp_ref[...]) + pgpb_ref[...]
    p1 = jnp.maximum(_dotT(z, ph1_ref[...]) + ph1b_ref[...], 0.0)
    pol_ref[...] = _dotT(p1, ph2_ref[...]) + ph2b_ref[...]
    v1 = jnp.maximum(_dotT(gr, vh1_ref[...]) + vh1b_ref[...], 0.0)
    v2 = jnp.maximum(_dotT(v1, vh2_ref[...]) + vh2b_ref[...], 0.0)
    val_ref[...] = jnp.tanh(_dotT(v2, vh3_ref[...]) + vh3b_ref[...])


_APAD = 6272
_PPAD = 8

_pool_call = pl.pallas_call(
    _pool_body,
    out_shape=(jax.ShapeDtypeStruct((1, _APAD), jnp.float32),
               jax.ShapeDtypeStruct((1, _PPAD), jnp.float32)))


def kernel(x, edge_index, enc_W, enc_b, ln0_g, ln0_b, Wl, bl, Wr, lng, lnb,
           A1, a1b, A2, a2b, pgp_W, pgp_b, ph1_W, ph1_b, ph2_W, ph2_b,
           vh1_W, vh1_b, vh2_W, vh2_b, vh3_W, vh3_b):
    src = edge_index[0].astype(jnp.int32)
    dst = edge_index[1].astype(jnp.int32)
    pad = EPAD - E
    src_p = jnp.concatenate([src, jnp.zeros((pad,), jnp.int32)])
    dst_p = jnp.concatenate([dst, jnp.full((pad,), N, jnp.int32)])

    zeros_h = jnp.zeros((NPAD, H), jnp.float32)
    zeros_c = jnp.zeros((NPAD, CW), jnp.float32)
    ones_c = jnp.ones((K, CW), jnp.float32)

    b2 = lambda v: v.reshape(1, -1)

    sc_agg_call, sc_counts_call = _sc_calls()
    h = _enc_call(x, enc_W, b2(enc_b), b2(ln0_g), b2(ln0_b))
    cnt = sc_counts_call(dst_p, ones_c, zeros_c)

    for i in range(L):
        aggp = sc_agg_call(h, src_p, dst_p, zeros_h)
        h = _layer_call(aggp, cnt, h, Wl[i], b2(bl[i]), Wr[i],
                        b2(lng[i]), b2(lnb[i]))

    ph2_Wp = jnp.zeros((_APAD, H), jnp.float32).at[:A].set(ph2_W)
    ph2_bp = jnp.zeros((1, _APAD), jnp.float32).at[:, :A].set(b2(ph2_b))
    vh3_Wp = jnp.zeros((_PPAD, H // 2), jnp.float32).at[:P].set(vh3_W)
    vh3_bp = jnp.zeros((1, _PPAD), jnp.float32).at[:, :P].set(b2(vh3_b))

    pol_p, val_p = _pool_call(
        h, A1, b2(a1b), A2,
        pgp_W[:, :H], b2(pgp_b), ph1_W, b2(ph1_b), ph2_Wp, ph2_bp,
        vh1_W[:, :H], b2(vh1_b), vh2_W, b2(vh2_b), vh3_Wp, vh3_bp)

    return pol_p[:, :A], val_p[:, :P]


# speedup vs baseline: 1.4253x; 1.0024x over previous
import functools

import jax
import jax.numpy as jnp
from jax import lax
from jax.experimental import pallas as pl
from jax.experimental.pallas import tpu as pltpu
from jax.experimental.pallas import tpu_sc as plsc

N = 10000
E = 320000
F_IN = 32
H = 128
L = 6
A = 6158
P = 4

NC = 2
NS = 16
NW = NC * NS

K = 128
EPW = ((E + NW * K - 1) // (NW * K)) * K
EPAD = EPW * NW
NCHUNK = EPW // K
NPAD = 10240
RPT = NPAD // NS
CW = 128

def _sc_agg(h_hbm, src_hbm, dst_hbm, zeros_hbm, out_hbm,
            src_v, dst_v, rows_v, acc_sh, sem):
    c = lax.axis_index("c")
    s = lax.axis_index("s")
    wid = s * NC + c

    @pl.when(s % 4 == 0)
    def _():
        q = (s // 4) * (NPAD // 4)
        pltpu.sync_copy(zeros_hbm.at[pl.ds(q, NPAD // 4)],
                        acc_sh.at[pl.ds(q, NPAD // 4)])

    plsc.subcore_barrier()

    base = wid * EPW

    def body(g, carry):
        off = base + g * K
        pltpu.sync_copy(src_hbm.at[pl.ds(off, K)], src_v)
        pltpu.sync_copy(dst_hbm.at[pl.ds(off, K)], dst_v)
        pltpu.async_copy(h_hbm.at[src_v], rows_v, sem).wait()
        pltpu.sync_copy(rows_v, acc_sh.at[dst_v], add=True)
        return carry

    lax.fori_loop(0, NCHUNK, body, 0)

    plsc.subcore_barrier()
    pltpu.sync_copy(acc_sh.at[pl.ds(s * RPT, RPT)],
                    out_hbm.at[c].at[pl.ds(s * RPT, RPT)])


def _sc_counts(dst_hbm, ones_hbm, zeros_hbm, out_hbm,
               dst_v, ones_v, acc_sh, sem):
    c = lax.axis_index("c")
    s = lax.axis_index("s")
    wid = s * NC + c

    @pl.when(s == 0)
    def _():
        pltpu.sync_copy(zeros_hbm, acc_sh)

    pltpu.async_copy(ones_hbm, ones_v, sem).wait()
    plsc.subcore_barrier()

    base = wid * EPW

    def body(g, carry):
        off = base + g * K
        pltpu.sync_copy(dst_hbm.at[pl.ds(off, K)], dst_v)
        pltpu.sync_copy(ones_v, acc_sh.at[dst_v], add=True)
        return carry

    lax.fori_loop(0, NCHUNK, body, 0)

    plsc.subcore_barrier()
    pltpu.sync_copy(acc_sh.at[pl.ds(s * RPT, RPT)],
                    out_hbm.at[c].at[pl.ds(s * RPT, RPT)])


@functools.lru_cache(maxsize=None)
def _sc_calls():
    mesh = plsc.VectorSubcoreMesh(core_axis_name="c", subcore_axis_name="s",
                                  num_cores=NC, num_subcores=NS)
    agg = pl.kernel(
        _sc_agg,
        mesh=mesh,
        out_type=jax.ShapeDtypeStruct((NC, NPAD, H), jnp.float32),
        scratch_types=[
            pltpu.VMEM((K,), jnp.int32),
            pltpu.VMEM((K,), jnp.int32),
            pltpu.VMEM((K, H), jnp.float32),
            pltpu.VMEM_SHARED((NPAD, H), jnp.float32),
            pltpu.SemaphoreType.DMA,
        ],
    )
    counts = pl.kernel(
        _sc_counts,
        mesh=mesh,
        out_type=jax.ShapeDtypeStruct((NC, NPAD, CW), jnp.float32),
        scratch_types=[
            pltpu.VMEM((K,), jnp.int32),
            pltpu.VMEM((K, CW), jnp.float32),
            pltpu.VMEM_SHARED((NPAD, CW), jnp.float32),
            pltpu.SemaphoreType.DMA,
        ],
    )
    return agg, counts


def _dotT(a, b):
    return lax.dot_general(a, b, (((1,), (1,)), ((), ())),
                           preferred_element_type=jnp.float32)


def _ln(t, g, b):
    m = jnp.mean(t, axis=-1, keepdims=True)
    v = jnp.mean((t - m) ** 2, axis=-1, keepdims=True)
    return (t - m) * lax.rsqrt(v + 1e-5) * g + b


def _enc_body(x_ref, w_ref, b_ref, g_ref, be_ref, out_ref):
    t = _dotT(x_ref[...], w_ref[...]) + b_ref[...]
    out_ref[...] = jnp.maximum(_ln(t, g_ref[...], be_ref[...]), 0.0)


_enc_call = pl.pallas_call(
    _enc_body, out_shape=jax.ShapeDtypeStruct((N, H), jnp.float32))


def _layer_body(aggp_ref, cnt_ref, h_ref, wl_ref, bl_ref, wr_ref,
                g_ref, b_ref, out_ref):
    cnt = cnt_ref[0, 0:N, 0:1] + cnt_ref[1, 0:N, 0:1]
    inv = 1.0 / jnp.maximum(cnt, 1.0)
    agg = (aggp_ref[0, 0:N, :] + aggp_ref[1, 0:N, :]) * inv
    hx = h_ref[...]
    t = _dotT(agg, wl_ref[...]) + bl_ref[...] + _dotT(hx, wr_ref[...])
    out_ref[...] = hx + jnp.maximum(_ln(t, g_ref[...], b_ref[...]), 0.0)


_layer_call = pl.pallas_call(
    _layer_body, out_shape=jax.ShapeDtypeStruct((N, H), jnp.float32))


def _pool_body(h_ref, a1_ref, a1b_ref, a2_ref,
               pgp_ref, pgpb_ref, ph1_ref, ph1b_ref, ph2_ref, ph2b_ref,
               vh1_ref, vh1b_ref, vh2_ref, vh2b_ref, vh3_ref, vh3b_ref,
               pol_ref, val_ref):
    hx = h_ref[...]
    t = jnp.tanh(_dotT(hx, a1_ref[...]) + a1b_ref[...])
    scores = _dotT(t, a2_ref[...])
    m = jnp.max(scores)
    e = jnp.exp(scores - m)
    w = e * (1.0 / jnp.sum(e))
    gr = jnp.sum(hx * w, axis=0, keepdims=True)
    z = _dotT(gr, pgp_ref[...]) + pgpb_ref[...]
    p1 = jnp.maximum(_dotT(z, ph1_ref[...]) + ph1b_ref[...], 0.0)
    pol_ref[...] = _dotT(p1, ph2_ref[...]) + ph2b_ref[...]
    v1 = jnp.maximum(_dotT(gr, vh1_ref[...]) + vh1b_ref[...], 0.0)
    v2 = jnp.maximum(_dotT(v1, vh2_ref[...]) + vh2b_ref[...], 0.0)
    val_ref[...] = jnp.tanh(_dotT(v2, vh3_ref[...]) + vh3b_ref[...])


_APAD = 6272
_PPAD = 8

_pool_call = pl.pallas_call(
    _pool_body,
    out_shape=(jax.ShapeDtypeStruct((1, _APAD), jnp.float32),
               jax.ShapeDtypeStruct((1, _PPAD), jnp.float32)))


def kernel(x, edge_index, enc_W, enc_b, ln0_g, ln0_b, Wl, bl, Wr, lng, lnb,
           A1, a1b, A2, a2b, pgp_W, pgp_b, ph1_W, ph1_b, ph2_W, ph2_b,
           vh1_W, vh1_b, vh2_W, vh2_b, vh3_W, vh3_b):
    src = edge_index[0].astype(jnp.int32)
    dst = edge_index[1].astype(jnp.int32)
    pad = EPAD - E
    src_p = jnp.concatenate([src, jnp.zeros((pad,), jnp.int32)])
    dst_p = jnp.concatenate([dst, jnp.full((pad,), N, jnp.int32)])

    zeros_h = jnp.zeros((NPAD, H), jnp.float32)
    zeros_c = jnp.zeros((NPAD, CW), jnp.float32)
    ones_c = jnp.ones((K, CW), jnp.float32)

    b2 = lambda v: v.reshape(1, -1)

    sc_agg_call, sc_counts_call = _sc_calls()
    h = _enc_call(x, enc_W, b2(enc_b), b2(ln0_g), b2(ln0_b))
    cnt = sc_counts_call(dst_p, ones_c, zeros_c)

    for i in range(L):
        aggp = sc_agg_call(h, src_p, dst_p, zeros_h)
        h = _layer_call(aggp, cnt, h, Wl[i], b2(bl[i]), Wr[i],
                        b2(lng[i]), b2(lnb[i]))

    ph2_Wp = jnp.zeros((_APAD, H), jnp.float32).at[:A].set(ph2_W)
    ph2_bp = jnp.zeros((1, _APAD), jnp.float32).at[:, :A].set(b2(ph2_b))
    vh3_Wp = jnp.zeros((_PPAD, H // 2), jnp.float32).at[:P].set(vh3_W)
    vh3_bp = jnp.zeros((1, _PPAD), jnp.float32).at[:, :P].set(b2(vh3_b))

    pol_p, val_p = _pool_call(
        h, A1, b2(a1b), A2,
        pgp_W[:, :H], b2(pgp_b), ph1_W, b2(ph1_b), ph2_Wp, ph2_bp,
        vh1_W[:, :H], b2(vh1_b), vh2_W, b2(vh2_b), vh3_Wp, vh3_bp)

    return pol_p[:, :A], val_p[:, :P]
